# baseline (speedup 1.0000x reference)
import functools

import jax
import jax.numpy as jnp
from jax import lax
from jax.experimental import pallas as pl
from jax.experimental.pallas import tpu as pltpu
from jax.experimental.pallas import tpu_sc as plsc

N_NODES = 10000
N_EDGES = 320000
D = 128
N_GRAPHS = 64

NC = 2
NS = 16
NW = NC * NS

CHUNK = 128
_CPT_MIN = (N_EDGES + NW * CHUNK - 1) // (NW * CHUNK)
CPT = _CPT_MIN + (_CPT_MIN % 2)
E_PAD = NW * CPT * CHUNK

ACC_ROWS = 10240
ZERO_ROWS_PER_TILE = ACC_ROWS // NS


def _segment_sum_sc(x, src_t, dst_t):
    mesh = plsc.VectorSubcoreMesh(core_axis_name="c", subcore_axis_name="s")

    @functools.partial(
        pl.kernel,
        out_type=jax.ShapeDtypeStruct((NC, ACC_ROWS, D), jnp.float32),
        mesh=mesh,
        scratch_types=[
            pltpu.VMEM_SHARED((ACC_ROWS, D), jnp.float32),
            pltpu.VMEM((2, CHUNK), jnp.int32),
            pltpu.VMEM((4, CHUNK), jnp.int32),
            pltpu.VMEM((CHUNK, D), jnp.float32),
            pltpu.VMEM((CHUNK, D), jnp.float32),
            pltpu.SemaphoreType.DMA,
            pltpu.SemaphoreType.DMA,
            pltpu.SemaphoreType.DMA,
            pltpu.SemaphoreType.DMA,
            pltpu.SemaphoreType.DMA,
            pltpu.SemaphoreType.DMA,
        ],
    )
    def seg_sum(x_hbm, src_hbm, dst_hbm, out_hbm, acc, sidx, didx, buf0, buf1,
                gsem0, gsem1, ssem0, ssem1, isem0, isem1):
        c = lax.axis_index("c")
        s = lax.axis_index("s")
        wid = c * NS + s

        zvec = jnp.zeros((16,), jnp.float32)

        @pl.loop(0, CHUNK)
        def _zero_rows(r):
            for cc in range(D // 16):
                buf0[r, pl.ds(cc * 16, 16)] = zvec

        off = 0
        while off < ZERO_ROWS_PER_TILE:
            n = min(CHUNK, ZERO_ROWS_PER_TILE - off)
            pltpu.sync_copy(
                buf0.at[pl.ds(0, n)],
                acc.at[pl.ds(s * ZERO_ROWS_PER_TILE + off, n)],
            )
            off += n
        plsc.subcore_barrier()

        isems = (isem0, isem1)
        gsems = (gsem0, gsem1)
        ssems = (ssem0, ssem1)

        def fire_idx(i, slot2, slot4):
            pltpu.async_copy(src_hbm.at[wid, i], sidx.at[slot2], isems[slot2])
            pltpu.async_copy(dst_hbm.at[wid, i], didx.at[slot4], isems[slot2])

        def drain_idx(slot2, slot4):
            pltpu.make_async_copy(src_hbm.at[wid, 0], sidx.at[slot2], isems[slot2]).wait()
            pltpu.make_async_copy(dst_hbm.at[wid, 0], didx.at[slot4], isems[slot2]).wait()

        def fire_rows(slot2, buf):
            pltpu.async_copy(x_hbm.at[sidx.at[slot2]], buf, gsems[slot2])

        def drain_rows(slot2, buf):
            pltpu.make_async_copy(x_hbm.at[sidx.at[slot2]], buf, gsems[slot2]).wait()

        def fire_scat(slot2, slot4, buf):
            pltpu.make_async_copy(buf, acc.at[didx.at[slot4]], ssems[slot2]).start(add=True)

        def drain_scat(slot2, slot4, buf):
            pltpu.make_async_copy(buf, acc.at[didx.at[slot4]], ssems[slot2]).wait()

        fire_idx(0, 0, 0)
        fire_idx(1, 1, 1)
        drain_idx(0, 0)
        fire_rows(0, buf0)
        drain_idx(1, 1)
        fire_rows(1, buf1)
        drain_rows(0, buf0)
        fire_scat(0, 0, buf0)
        fire_idx(2, 0, 2)
        drain_rows(1, buf1)
        fire_scat(1, 1, buf1)
        fire_idx(3, 1, 3)
        drain_idx(0, 2)
        drain_scat(0, 0, buf0)
        fire_rows(0, buf0)

        @pl.loop(1, CPT // 2 - 1)
        def _edges(j):
            i = j * 2
            i1 = (i + 1) % 4
            i2 = (i + 2) % 4
            i3 = (i + 3) % 4
            drain_idx(1, i1)
            drain_scat(1, (i - 1) % 4, buf1)
            fire_rows(1, buf1)
            drain_rows(0, buf0)
            fire_scat(0, i % 4, buf0)
            fire_idx(i + 2, 0, i2)
            drain_rows(1, buf1)
            fire_scat(1, i1, buf1)
            fire_idx(i + 3, 1, i3)
            drain_idx(0, i2)
            drain_scat(0, i % 4, buf0)
            fire_rows(0, buf0)

        iL = (CPT - 1) % 4
        drain_idx(1, iL)
        drain_scat(1, (CPT - 3) % 4, buf1)
        fire_rows(1, buf1)
        drain_rows(0, buf0)
        fire_scat(0, (CPT - 2) % 4, buf0)
        drain_rows(1, buf1)
        fire_scat(1, iL, buf1)
        drain_scat(0, (CPT - 2) % 4, buf0)
        drain_scat(1, iL, buf1)

        plsc.subcore_barrier()

        pltpu.sync_copy(
            acc.at[pl.ds(s * ZERO_ROWS_PER_TILE, ZERO_ROWS_PER_TILE)],
            out_hbm.at[c, pl.ds(s * ZERO_ROWS_PER_TILE, ZERO_ROWS_PER_TILE)],
        )

    return seg_sum(x, src_t, dst_t)


def _dense_layer_tc(parts, x, w_rel, b, w_root):
    def body(p_ref, x_ref, wr_ref, b_ref, wq_ref, o_ref):
        agg = p_ref[0, :N_NODES] + p_ref[1, :N_NODES]
        h = jnp.dot(agg, wr_ref[...], preferred_element_type=jnp.float32)
        h = h + jnp.dot(x_ref[...], wq_ref[...], preferred_element_type=jnp.float32)
        h = h + b_ref[...]
        o_ref[...] = jnp.maximum(h, 0.0)

    return pl.pallas_call(
        body,
        out_shape=jax.ShapeDtypeStruct((N_NODES, D), jnp.float32),
    )(parts, x, w_rel, b.reshape(1, D), w_root)


def _final_tc(parts, h, w_rel, b, w_root, batch2d, fc_w, fc_b):
    def body(p_ref, h_ref, wr_ref, b_ref, wq_ref, bt_ref, fw_ref, fb_ref, o_ref):
        agg = p_ref[0, :N_NODES] + p_ref[1, :N_NODES]
        h2 = jnp.dot(agg, wr_ref[...], preferred_element_type=jnp.float32)
        h2 = h2 + jnp.dot(h_ref[...], wq_ref[...], preferred_element_type=jnp.float32)
        h2 = jnp.maximum(h2 + b_ref[...], 0.0)
        gids = lax.broadcasted_iota(jnp.int32, (N_GRAPHS, N_NODES), 0)
        sel = (gids == bt_ref[...]).astype(jnp.float32)
        sums = jnp.dot(sel, h2, preferred_element_type=jnp.float32)
        counts = jnp.sum(sel, axis=1, keepdims=True)
        pooled = sums / jnp.maximum(counts, 1.0)
        out = jnp.dot(pooled, fw_ref[...], preferred_element_type=jnp.float32)
        o_ref[...] = out + fb_ref[...]

    return pl.pallas_call(
        body,
        out_shape=jax.ShapeDtypeStruct((N_GRAPHS, 10), jnp.float32),
    )(parts, h, w_rel, b.reshape(1, D), w_root, batch2d, fc_w, fc_b.reshape(1, 10))


def kernel(x, edge_index, batch, W1_rel, b1, W1_root, W2_rel, b2, W2_root, fc_W, fc_b):
    x = x.astype(jnp.float32)
    src = edge_index[0].astype(jnp.int32)
    dst = edge_index[1].astype(jnp.int32)

    n_pad = E_PAD - N_EDGES
    pad_ids = jnp.arange(n_pad, dtype=jnp.int32)
    pad_src = (pad_ids * 97) % N_NODES
    pad_dst = N_NODES + pad_ids % (ACC_ROWS - N_NODES)
    src_t = jnp.concatenate([src, pad_src]).reshape(NW, CPT, CHUNK)
    dst_t = jnp.concatenate([dst, pad_dst]).reshape(NW, CPT, CHUNK)

    p1 = _segment_sum_sc(x, src_t, dst_t)
    h = _dense_layer_tc(p1, x, W1_rel, b1, W1_root)
    p2 = _segment_sum_sc(h, src_t, dst_t)
    batch2d = batch.astype(jnp.int32).reshape(1, N_NODES)
    return _final_tc(p2, h, W2_rel, b2, W2_root, batch2d, fc_W, fc_b)
